# Initial kernel scaffold; baseline (speedup 1.0000x reference)
import jax
import jax.numpy as jnp
from jax.experimental import pallas as pl
from jax.experimental.pallas import tpu as pltpu
from jax.experimental.pallas import tpu_sc as plsc

WINDOW = 128

_mesh = plsc.VectorSubcoreMesh(core_axis_name="core", subcore_axis_name="subcore")


def _gather_flat(x_flat, embeddings):
    n = x_flat.shape[0]
    d = embeddings.shape[1]

    @pl.kernel(
        out_type=jax.ShapeDtypeStruct((n, d), embeddings.dtype),
        mesh=_mesh,
    )
    def gather_kernel(emb_hbm, idx_hbm, out_hbm):
        def body(idx_vmem, out_vmem):
            pltpu.sync_copy(emb_hbm.at[idx_vmem.at[0]], out_vmem)

        pltpu.emit_pipeline(
            body,
            grid=(n // WINDOW,),
            in_specs=[pl.BlockSpec((1, WINDOW), index_map=lambda i: (0, i))],
            out_specs=[pl.BlockSpec((WINDOW, d), index_map=lambda i: (i, 0))],
            core_axis_name=("core", "subcore"),
            dimension_semantics=(pltpu.PARALLEL,),
        )(idx_hbm, out_hbm)

    return gather_kernel(embeddings, x_flat.reshape(1, n))


def kernel(x, embeddings):
    batch, hist = x.shape
    out = _gather_flat(x.reshape(-1), embeddings)
    return out.reshape(batch, hist, embeddings.shape[1])


# speedup vs baseline: 1.8104x; 1.8104x over previous
import functools

import jax
import jax.numpy as jnp
from jax import lax
from jax.experimental import pallas as pl
from jax.experimental.pallas import tpu as pltpu
from jax.experimental.pallas import tpu_sc as plsc

NC = 2
NS = 16
NW = NC * NS

CHUNK = 128

_mesh = plsc.VectorSubcoreMesh(core_axis_name="c", subcore_axis_name="s")


def _gather_flat(x_flat, embeddings):
    n = x_flat.shape[0]
    d = embeddings.shape[1]
    assert n % (NW * 2 * CHUNK) == 0
    b_per_w = n // NW
    n_pairs = b_per_w // (2 * CHUNK)

    @functools.partial(
        pl.kernel,
        mesh=_mesh,
        compiler_params=pltpu.CompilerParams(use_tc_tiling_on_sc=False),
        out_type=jax.ShapeDtypeStruct((n, d), embeddings.dtype),
        scratch_types=[
            pltpu.VMEM((b_per_w,), jnp.int32),
            pltpu.VMEM((CHUNK, d), jnp.float32),
            pltpu.VMEM((CHUNK, d), jnp.float32),
            pltpu.SemaphoreType.DMA,
            pltpu.SemaphoreType.DMA,
            pltpu.SemaphoreType.DMA,
            pltpu.SemaphoreType.DMA,
        ],
    )
    def gather_kernel(emb_hbm, idx_hbm, out_hbm, idx_v, rows_a, rows_b,
                      gsem_a, gsem_b, osem_a, osem_b):
        wid = lax.axis_index("s") * NC + lax.axis_index("c")
        base = wid * b_per_w
        pltpu.sync_copy(idx_hbm.at[pl.ds(base, b_per_w)], idx_v)

        def gather_start(c, rows_v, gsem):
            pltpu.async_copy(emb_hbm.at[idx_v.at[pl.ds(c, CHUNK)]], rows_v, gsem)

        def gather_wait(rows_v, gsem):
            pltpu.make_async_copy(emb_hbm.at[idx_v.at[pl.ds(0, CHUNK)]], rows_v,
                                  gsem).wait()

        def out_start(c, rows_v, osem):
            pltpu.async_copy(rows_v, out_hbm.at[pl.ds(base + c, CHUNK)], osem)

        def out_wait(rows_v, osem):
            pltpu.make_async_copy(rows_v, out_hbm.at[pl.ds(base, CHUNK)],
                                  osem).wait()

        gather_start(0, rows_a, gsem_a)
        gather_start(CHUNK, rows_b, gsem_b)

        @pl.loop(0, n_pairs - 1)
        def _(i):
            c0 = i * (2 * CHUNK)
            gather_wait(rows_a, gsem_a)
            out_start(c0, rows_a, osem_a)
            gather_wait(rows_b, gsem_b)
            out_start(c0 + CHUNK, rows_b, osem_b)
            out_wait(rows_a, osem_a)
            gather_start(c0 + 2 * CHUNK, rows_a, gsem_a)
            out_wait(rows_b, osem_b)
            gather_start(c0 + 3 * CHUNK, rows_b, gsem_b)

        c_last = (n_pairs - 1) * (2 * CHUNK)
        gather_wait(rows_a, gsem_a)
        out_start(c_last, rows_a, osem_a)
        gather_wait(rows_b, gsem_b)
        out_start(c_last + CHUNK, rows_b, osem_b)
        out_wait(rows_a, osem_a)
        out_wait(rows_b, osem_b)

    return gather_kernel(embeddings, x_flat)


def kernel(x, embeddings):
    batch, hist = x.shape
    out = _gather_flat(x.reshape(-1), embeddings)
    return out.reshape(batch, hist, embeddings.shape[1])


# speedup vs baseline: 1.8592x; 1.0269x over previous
import functools

import jax
import jax.numpy as jnp
from jax import lax
from jax.experimental import pallas as pl
from jax.experimental.pallas import tpu as pltpu
from jax.experimental.pallas import tpu_sc as plsc

NC = 2
NS = 16
NW = NC * NS

G = 8

_mesh = plsc.VectorSubcoreMesh(core_axis_name="c", subcore_axis_name="s")


def _gather(x, embeddings):
    batch, hist = x.shape
    d = embeddings.shape[1]
    assert batch % (NW * 2 * G) == 0
    rows_per_w = batch // NW
    n_pairs = rows_per_w // (2 * G)

    @functools.partial(
        pl.kernel,
        mesh=_mesh,
        compiler_params=pltpu.CompilerParams(use_tc_tiling_on_sc=False),
        out_type=jax.ShapeDtypeStruct((batch, hist, d), embeddings.dtype),
        scratch_types=[
            pltpu.VMEM((rows_per_w, hist), jnp.int32),
            pltpu.VMEM((G, hist, d), jnp.float32),
            pltpu.VMEM((G, hist, d), jnp.float32),
            pltpu.SemaphoreType.DMA,
            pltpu.SemaphoreType.DMA,
            pltpu.SemaphoreType.DMA,
            pltpu.SemaphoreType.DMA,
        ],
    )
    def gather_kernel(emb_hbm, idx_hbm, out_hbm, idx_v, rows_a, rows_b,
                      gsem_a, gsem_b, osem_a, osem_b):
        wid = lax.axis_index("s") * NC + lax.axis_index("c")
        row0 = wid * rows_per_w
        pltpu.sync_copy(idx_hbm.at[pl.ds(row0, rows_per_w)], idx_v)

        def gather_start(g, rows_v, gsem):
            for j in range(G):
                pltpu.async_copy(emb_hbm.at[idx_v.at[g * G + j]], rows_v.at[j],
                                 gsem)

        def gather_wait(rows_v, gsem):
            for j in range(G):
                pltpu.make_async_copy(emb_hbm.at[idx_v.at[0]], rows_v.at[j],
                                      gsem).wait()

        def out_start(g, rows_v, osem):
            pltpu.async_copy(rows_v, out_hbm.at[pl.ds(row0 + g * G, G)], osem)

        def out_wait(rows_v, osem):
            pltpu.make_async_copy(rows_v, out_hbm.at[pl.ds(row0, G)],
                                  osem).wait()

        gather_start(0, rows_a, gsem_a)
        gather_start(1, rows_b, gsem_b)

        @pl.loop(0, n_pairs - 1)
        def _(i):
            g0 = i * 2
            gather_wait(rows_a, gsem_a)
            out_start(g0, rows_a, osem_a)
            gather_wait(rows_b, gsem_b)
            out_start(g0 + 1, rows_b, osem_b)
            out_wait(rows_a, osem_a)
            gather_start(g0 + 2, rows_a, gsem_a)
            out_wait(rows_b, osem_b)
            gather_start(g0 + 3, rows_b, gsem_b)

        g_last = (n_pairs - 1) * 2
        gather_wait(rows_a, gsem_a)
        out_start(g_last, rows_a, osem_a)
        gather_wait(rows_b, gsem_b)
        out_start(g_last + 1, rows_b, osem_b)
        out_wait(rows_a, osem_a)
        out_wait(rows_b, osem_b)

    return gather_kernel(embeddings, x)


def kernel(x, embeddings):
    return _gather(x, embeddings)


# speedup vs baseline: 1.8595x; 1.0002x over previous
import functools

import jax
import jax.numpy as jnp
from jax import lax
from jax.experimental import pallas as pl
from jax.experimental.pallas import tpu as pltpu
from jax.experimental.pallas import tpu_sc as plsc

NC = 2
NS = 16
NW = NC * NS

G = 8
IDX_COLS = 64
GW = 56
PAD_IDX = 2**30

_mesh = plsc.VectorSubcoreMesh(core_axis_name="c", subcore_axis_name="s")


def _gather(x_pad, hist, embeddings):
    batch = x_pad.shape[0]
    d = embeddings.shape[1]
    assert batch % (NW * 2 * G) == 0
    rows_per_w = batch // NW
    n_pairs = rows_per_w // (2 * G)

    @functools.partial(
        pl.kernel,
        mesh=_mesh,
        compiler_params=pltpu.CompilerParams(use_tc_tiling_on_sc=False),
        out_type=jax.ShapeDtypeStruct((batch, hist, d), embeddings.dtype),
        scratch_types=[
            pltpu.VMEM((rows_per_w, IDX_COLS), jnp.int32),
            pltpu.VMEM((G, GW, d), jnp.float32),
            pltpu.VMEM((G, GW, d), jnp.float32),
            pltpu.SemaphoreType.DMA,
            pltpu.SemaphoreType.DMA,
            pltpu.SemaphoreType.DMA,
            pltpu.SemaphoreType.DMA,
        ],
    )
    def gather_kernel(emb_hbm, idx_hbm, out_hbm, idx_v, rows_a, rows_b,
                      gsem_a, gsem_b, osem_a, osem_b):
        wid = lax.axis_index("s") * NC + lax.axis_index("c")
        row0 = wid * rows_per_w
        pltpu.sync_copy(idx_hbm.at[pl.ds(row0, rows_per_w), pl.ds(0, IDX_COLS)],
                        idx_v)

        def row_indices(r):
            return plsc.Indices(idx_v.at[r, pl.ds(0, GW)],
                                ignored_value=PAD_IDX)

        def gather_start(g, rows_v, gsem):
            for j in range(G):
                pltpu.async_copy(emb_hbm.at[row_indices(g * G + j)],
                                 rows_v.at[j], gsem)

        def gather_wait(rows_v, gsem):
            for j in range(G):
                pltpu.make_async_copy(emb_hbm.at[row_indices(0)],
                                      rows_v.at[j], gsem).wait()

        def out_start(g, rows_v, osem):
            pltpu.async_copy(rows_v.at[pl.ds(0, G), pl.ds(0, hist)],
                             out_hbm.at[pl.ds(row0 + g * G, G)], osem)

        def out_wait(rows_v, osem):
            pltpu.make_async_copy(rows_v.at[pl.ds(0, G), pl.ds(0, hist)],
                                  out_hbm.at[pl.ds(row0, G)], osem).wait()

        gather_start(0, rows_a, gsem_a)
        gather_start(1, rows_b, gsem_b)

        @pl.loop(0, n_pairs - 1)
        def _(i):
            g0 = i * 2
            gather_wait(rows_a, gsem_a)
            out_start(g0, rows_a, osem_a)
            gather_wait(rows_b, gsem_b)
            out_start(g0 + 1, rows_b, osem_b)
            out_wait(rows_a, osem_a)
            gather_start(g0 + 2, rows_a, gsem_a)
            out_wait(rows_b, osem_b)
            gather_start(g0 + 3, rows_b, gsem_b)

        g_last = (n_pairs - 1) * 2
        gather_wait(rows_a, gsem_a)
        out_start(g_last, rows_a, osem_a)
        gather_wait(rows_b, gsem_b)
        out_start(g_last + 1, rows_b, osem_b)
        out_wait(rows_a, osem_a)
        out_wait(rows_b, osem_b)

    return gather_kernel(embeddings, x_pad)


def kernel(x, embeddings):
    batch, hist = x.shape
    x_pad = jnp.pad(x, ((0, 0), (0, 128 - hist)), constant_values=PAD_IDX)
    return _gather(x_pad, hist, embeddings)
